# Initial kernel scaffold; baseline (speedup 1.0000x reference)
import jax
import jax.numpy as jnp
from jax.experimental import pallas as pl


def kernel(x, edge_index, edge_attr, Wv, bv, We, be, WG, u, Wr1, br1, Wr2, br2, Wout, bout):
    raise NotImplementedError("write your pallas kernel here")


# speedup vs baseline: 2.2270x; 2.2270x over previous
import functools

import jax
import jax.numpy as jnp
from jax import lax
from jax.experimental import pallas as pl
from jax.experimental.pallas import tpu as pltpu
from jax.experimental.pallas import tpu_sc as plsc

N = 10000
E = 320000
F = 128
K = 16

NC = 2
NS = 16
NW = NC * NS
EPW = E // NW
C = 80
NCHUNK = EPW // C
NPAD = 10240
RPS = NPAD // NS
ZR = 128

_NBLK = 1000
_EBLK = 8000


def _dot_t(a, w):
    return lax.dot_general(a, w, (((1,), (1,)), ((), ())),
                           precision=lax.Precision.HIGHEST,
                           preferred_element_type=jnp.float32)


def _node_body(x_ref, we_ref, be_ref, wv_ref, bv_ref, he_ref, hv_ref):
    xa = jnp.maximum(x_ref[...], 0.0)
    he = _dot_t(xa, we_ref[...]) + be_ref[...]
    he_ref[...] = jnp.maximum(he, 0.0)
    hv = _dot_t(xa, wv_ref[...]) + bv_ref[...]
    hv_ref[...] = jnp.maximum(hv, 0.0)


def _node_call(x, We, be, Wv, bv):
    grid = (N // _NBLK,)
    return pl.pallas_call(
        _node_body,
        grid=grid,
        in_specs=[
            pl.BlockSpec((_NBLK, F), lambda i: (i, 0)),
            pl.BlockSpec((F, F), lambda i: (0, 0)),
            pl.BlockSpec((1, F), lambda i: (0, 0)),
            pl.BlockSpec((F, F), lambda i: (0, 0)),
            pl.BlockSpec((1, F), lambda i: (0, 0)),
        ],
        out_specs=[
            pl.BlockSpec((_NBLK, F), lambda i: (i, 0)),
            pl.BlockSpec((_NBLK, F), lambda i: (i, 0)),
        ],
        out_shape=[
            jax.ShapeDtypeStruct((N, F), jnp.float32),
            jax.ShapeDtypeStruct((N, F), jnp.float32),
        ],
    )(x, We, be.reshape(1, F), Wv, bv.reshape(1, F))


def _gate_body(ea_ref, wg_ref, gate_ref):
    gate_ref[...] = _dot_t(ea_ref[...], wg_ref[...])


def _gate_call(edge_attr, WG):
    grid = (E // _EBLK,)
    return pl.pallas_call(
        _gate_body,
        grid=grid,
        in_specs=[
            pl.BlockSpec((_EBLK, K), lambda i: (i, 0)),
            pl.BlockSpec((F, K), lambda i: (0, 0)),
        ],
        out_specs=pl.BlockSpec((_EBLK, F), lambda i: (i, 0)),
        out_shape=jax.ShapeDtypeStruct((E, F), jnp.float32),
    )(edge_attr, WG)


def _edge_sc_body(h_hbm, gate_hbm, src_hbm, dst_hbm, out_hbm,
                  src_v, dst_v, rows_v, gate_v, zbuf_v, acc_sh, sem):
    core = lax.axis_index("core")
    sid = lax.axis_index("subcore")
    wid = sid * NC + core

    @pl.loop(0, ZR)
    def _(r):
        for j in range(F // 16):
            zbuf_v[r, pl.ds(j * 16, 16)] = jnp.zeros((16,), jnp.float32)

    @pl.loop(0, RPS // ZR)
    def _(k):
        pltpu.sync_copy(zbuf_v, acc_sh.at[pl.ds(sid * RPS + k * ZR, ZR)])

    plsc.subcore_barrier()

    @pl.loop(0, NCHUNK)
    def _(i):
        base = wid * EPW + i * C
        pltpu.sync_copy(src_hbm.at[pl.ds(base, C)], src_v)
        pltpu.async_copy(h_hbm.at[src_v], rows_v, sem).wait()
        pltpu.sync_copy(gate_hbm.at[pl.ds(base, C)], gate_v)
        pltpu.sync_copy(dst_hbm.at[pl.ds(base, C)], dst_v)

        @pl.loop(0, C)
        def _(r):
            for j in range(F // 16):
                sl = pl.ds(j * 16, 16)
                rows_v[r, sl] = rows_v[r, sl] * gate_v[r, sl]

        pltpu.sync_copy(rows_v, acc_sh.at[dst_v], add=True)

    plsc.subcore_barrier()

    pltpu.sync_copy(acc_sh.at[pl.ds(sid * RPS, RPS)],
                    out_hbm.at[core, pl.ds(sid * RPS, RPS)])


def _edge_sc_call(h_e, gate, src, dst):
    mesh = plsc.VectorSubcoreMesh(core_axis_name="core",
                                  subcore_axis_name="subcore")
    k = pl.kernel(
        _edge_sc_body,
        out_type=jax.ShapeDtypeStruct((NC, NPAD, F), jnp.float32),
        mesh=mesh,
        scratch_types=[
            pltpu.VMEM((C,), jnp.int32),
            pltpu.VMEM((C,), jnp.int32),
            pltpu.VMEM((C, F), jnp.float32),
            pltpu.VMEM((C, F), jnp.float32),
            pltpu.VMEM((ZR, F), jnp.float32),
            pltpu.VMEM_SHARED((NPAD, F), jnp.float32),
            pltpu.SemaphoreType.DMA,
        ],
    )
    return k(h_e, gate, src, dst)


def _post_body(p_ref, hv_ref, x_ref, u_ref, wr1_ref, br1_ref, wr2_ref,
               br2_ref, wout_ref, bout_ref, out1_ref, out2_ref):
    aggr = p_ref[0] + p_ref[1]
    msgx = hv_ref[...] + aggr
    out2_ref[...] = msgx
    tmp = msgx
    for i in range(2):
        h = jnp.maximum(tmp, 0.0)
        h = jnp.maximum(_dot_t(h, wr1_ref[i]) + br1_ref[i], 0.0)
        h = _dot_t(h, wr2_ref[i]) + br2_ref[i]
        tmp = tmp + h
    v = _dot_t(tmp, wout_ref[...]) + bout_ref[...]
    out1_ref[...] = v + x_ref[...] * u_ref[...]


def _post_call(partials, h_v, x, u, Wr1, br1, Wr2, br2, Wout, bout):
    grid = (N // _NBLK,)
    return pl.pallas_call(
        _post_body,
        grid=grid,
        in_specs=[
            pl.BlockSpec((NC, _NBLK, F), lambda i: (0, i, 0)),
            pl.BlockSpec((_NBLK, F), lambda i: (i, 0)),
            pl.BlockSpec((_NBLK, F), lambda i: (i, 0)),
            pl.BlockSpec((1, F), lambda i: (0, 0)),
            pl.BlockSpec((2, F, F), lambda i: (0, 0, 0)),
            pl.BlockSpec((2, 1, F), lambda i: (0, 0, 0)),
            pl.BlockSpec((2, F, F), lambda i: (0, 0, 0)),
            pl.BlockSpec((2, 1, F), lambda i: (0, 0, 0)),
            pl.BlockSpec((F, F), lambda i: (0, 0)),
            pl.BlockSpec((1, F), lambda i: (0, 0)),
        ],
        out_specs=[
            pl.BlockSpec((_NBLK, F), lambda i: (i, 0)),
            pl.BlockSpec((_NBLK, F), lambda i: (i, 0)),
        ],
        out_shape=[
            jax.ShapeDtypeStruct((N, F), jnp.float32),
            jax.ShapeDtypeStruct((N, F), jnp.float32),
        ],
    )(partials, h_v, x, u, Wr1, br1.reshape(2, 1, F), Wr2,
      br2.reshape(2, 1, F), Wout, bout.reshape(1, F))


def kernel(x, edge_index, edge_attr, Wv, bv, We, be, WG, u, Wr1, br1, Wr2,
           br2, Wout, bout):
    src = edge_index[0]
    dst = edge_index[1]
    h_e, h_v = _node_call(x, We, be, Wv, bv)
    gate = _gate_call(edge_attr, WG)
    partials = _edge_sc_call(h_e, gate, src, dst)[:, :N, :]
    out1, msgx = _post_call(partials, h_v, x, u, Wr1, br1, Wr2, br2, Wout,
                            bout)
    return (out1, msgx)


# speedup vs baseline: 3.4183x; 1.5349x over previous
import functools

import jax
import jax.numpy as jnp
from jax import lax
from jax.experimental import pallas as pl
from jax.experimental.pallas import tpu as pltpu
from jax.experimental.pallas import tpu_sc as plsc

N = 10000
E = 320000
F = 128
K = 16

NC = 2
NS = 16
NW = NC * NS
EPW = E // NW
C = 40
NCHUNK = EPW // C
NPAD = 10240
RPS = NPAD // NS
ZR = 128

_NBLK = 1000
_EBLK = 8000


def _dot_t(a, w):
    return lax.dot_general(a, w, (((1,), (1,)), ((), ())),
                           precision=lax.Precision.HIGHEST,
                           preferred_element_type=jnp.float32)


def _node_body(x_ref, we_ref, be_ref, wv_ref, bv_ref, he_ref, hv_ref):
    xa = jnp.maximum(x_ref[...], 0.0)
    he = _dot_t(xa, we_ref[...]) + be_ref[...]
    he_ref[...] = jnp.maximum(he, 0.0)
    hv = _dot_t(xa, wv_ref[...]) + bv_ref[...]
    hv_ref[...] = jnp.maximum(hv, 0.0)


def _node_call(x, We, be, Wv, bv):
    grid = (N // _NBLK,)
    return pl.pallas_call(
        _node_body,
        grid=grid,
        in_specs=[
            pl.BlockSpec((_NBLK, F), lambda i: (i, 0)),
            pl.BlockSpec((F, F), lambda i: (0, 0)),
            pl.BlockSpec((1, F), lambda i: (0, 0)),
            pl.BlockSpec((F, F), lambda i: (0, 0)),
            pl.BlockSpec((1, F), lambda i: (0, 0)),
        ],
        out_specs=[
            pl.BlockSpec((_NBLK, F), lambda i: (i, 0)),
            pl.BlockSpec((_NBLK, F), lambda i: (i, 0)),
        ],
        out_shape=[
            jax.ShapeDtypeStruct((N, F), jnp.float32),
            jax.ShapeDtypeStruct((N, F), jnp.float32),
        ],
    )(x, We, be.reshape(1, F), Wv, bv.reshape(1, F))


def _gate_body(ea_ref, wg_ref, gate_ref):
    gate_ref[...] = _dot_t(ea_ref[...], wg_ref[...])


def _gate_call(edge_attr, WG):
    grid = (E // _EBLK,)
    return pl.pallas_call(
        _gate_body,
        grid=grid,
        in_specs=[
            pl.BlockSpec((_EBLK, K), lambda i: (i, 0)),
            pl.BlockSpec((F, K), lambda i: (0, 0)),
        ],
        out_specs=pl.BlockSpec((_EBLK, F), lambda i: (i, 0)),
        out_shape=jax.ShapeDtypeStruct((E, F), jnp.float32),
    )(edge_attr, WG)


def _edge_sc_body(h_hbm, gate_hbm, src_hbm, dst_hbm, out_hbm,
                  src_all, dst_all, rows0, rows1, gate0, gate1,
                  acc_sh, sem_g0, sem_g1, sem_r0, sem_r1):
    core = lax.axis_index("core")
    sid = lax.axis_index("subcore")
    wid = sid * NC + core

    rows = (rows0, rows1)
    gate = (gate0, gate1)
    sem_g = (sem_g0, sem_g1)
    sem_r = (sem_r0, sem_r1)

    for buf in (*rows, *gate):
        @pl.loop(0, C)
        def _(r, buf=buf):
            for j in range(F // 16):
                buf[r, pl.ds(j * 16, 16)] = jnp.zeros((16,), jnp.float32)

    @pl.loop(0, RPS // (4 * C))
    def _(k):
        for j, buf in enumerate((*rows, *gate)):
            pltpu.sync_copy(
                buf, acc_sh.at[pl.ds(sid * RPS + (k * 4 + j) * C, C)])

    pltpu.sync_copy(src_hbm.at[pl.ds(wid * EPW, EPW)], src_all)
    pltpu.sync_copy(dst_hbm.at[pl.ds(wid * EPW, EPW)], dst_all)

    plsc.subcore_barrier()

    def issue(i, b):
        base = wid * EPW + i * C
        pltpu.async_copy(gate_hbm.at[pl.ds(base, C)], gate[b], sem_g[b])
        pltpu.async_copy(h_hbm.at[src_all.at[pl.ds(i * C, C)]], rows[b],
                         sem_r[b])

    def consume(i, b):
        pltpu.make_async_copy(gate_hbm.at[pl.ds(0, C)], gate[b],
                              sem_g[b]).wait()
        pltpu.make_async_copy(h_hbm.at[src_all.at[pl.ds(0, C)]], rows[b],
                              sem_r[b]).wait()

        @pl.loop(0, C)
        def _(r):
            for j in range(F // 16):
                sl = pl.ds(j * 16, 16)
                rows[b][r, sl] = rows[b][r, sl] * gate[b][r, sl]

        pltpu.sync_copy(rows[b], acc_sh.at[dst_all.at[pl.ds(i * C, C)]],
                        add=True)

    issue(0, 0)
    issue(1, 1)

    @pl.loop(0, NCHUNK // 2 - 1)
    def _(t):
        i0 = 2 * t
        consume(i0, 0)
        issue(i0 + 2, 0)
        consume(i0 + 1, 1)
        issue(i0 + 3, 1)

    consume(NCHUNK - 2, 0)
    consume(NCHUNK - 1, 1)

    plsc.subcore_barrier()

    pltpu.sync_copy(acc_sh.at[pl.ds(sid * RPS, RPS)],
                    out_hbm.at[core, pl.ds(sid * RPS, RPS)])


def _edge_sc_call(h_e, gate, src, dst):
    mesh = plsc.VectorSubcoreMesh(core_axis_name="core",
                                  subcore_axis_name="subcore")
    k = pl.kernel(
        _edge_sc_body,
        out_type=jax.ShapeDtypeStruct((NC, NPAD, F), jnp.float32),
        mesh=mesh,
        scratch_types=[
            pltpu.VMEM((EPW,), jnp.int32),
            pltpu.VMEM((EPW,), jnp.int32),
            pltpu.VMEM((C, F), jnp.float32),
            pltpu.VMEM((C, F), jnp.float32),
            pltpu.VMEM((C, F), jnp.float32),
            pltpu.VMEM((C, F), jnp.float32),
            pltpu.VMEM_SHARED((NPAD, F), jnp.float32),
            pltpu.SemaphoreType.DMA,
            pltpu.SemaphoreType.DMA,
            pltpu.SemaphoreType.DMA,
            pltpu.SemaphoreType.DMA,
        ],
    )
    return k(h_e, gate, src, dst)


def _post_body(p_ref, hv_ref, x_ref, u_ref, wr1_ref, br1_ref, wr2_ref,
               br2_ref, wout_ref, bout_ref, out1_ref, out2_ref):
    aggr = p_ref[0] + p_ref[1]
    msgx = hv_ref[...] + aggr
    out2_ref[...] = msgx
    tmp = msgx
    for i in range(2):
        h = jnp.maximum(tmp, 0.0)
        h = jnp.maximum(_dot_t(h, wr1_ref[i]) + br1_ref[i], 0.0)
        h = _dot_t(h, wr2_ref[i]) + br2_ref[i]
        tmp = tmp + h
    v = _dot_t(tmp, wout_ref[...]) + bout_ref[...]
    out1_ref[...] = v + x_ref[...] * u_ref[...]


def _post_call(partials, h_v, x, u, Wr1, br1, Wr2, br2, Wout, bout):
    grid = (N // _NBLK,)
    return pl.pallas_call(
        _post_body,
        grid=grid,
        in_specs=[
            pl.BlockSpec((NC, _NBLK, F), lambda i: (0, i, 0)),
            pl.BlockSpec((_NBLK, F), lambda i: (i, 0)),
            pl.BlockSpec((_NBLK, F), lambda i: (i, 0)),
            pl.BlockSpec((1, F), lambda i: (0, 0)),
            pl.BlockSpec((2, F, F), lambda i: (0, 0, 0)),
            pl.BlockSpec((2, 1, F), lambda i: (0, 0, 0)),
            pl.BlockSpec((2, F, F), lambda i: (0, 0, 0)),
            pl.BlockSpec((2, 1, F), lambda i: (0, 0, 0)),
            pl.BlockSpec((F, F), lambda i: (0, 0)),
            pl.BlockSpec((1, F), lambda i: (0, 0)),
        ],
        out_specs=[
            pl.BlockSpec((_NBLK, F), lambda i: (i, 0)),
            pl.BlockSpec((_NBLK, F), lambda i: (i, 0)),
        ],
        out_shape=[
            jax.ShapeDtypeStruct((N, F), jnp.float32),
            jax.ShapeDtypeStruct((N, F), jnp.float32),
        ],
    )(partials, h_v, x, u, Wr1, br1.reshape(2, 1, F), Wr2,
      br2.reshape(2, 1, F), Wout, bout.reshape(1, F))


def kernel(x, edge_index, edge_attr, Wv, bv, We, be, WG, u, Wr1, br1, Wr2,
           br2, Wout, bout):
    src = edge_index[0]
    dst = edge_index[1]
    h_e, h_v = _node_call(x, We, be, Wv, bv)
    gate = _gate_call(edge_attr, WG)
    partials = _edge_sc_call(h_e, gate, src, dst)[:, :N, :]
    out1, msgx = _post_call(partials, h_v, x, u, Wr1, br1, Wr2, br2, Wout,
                            bout)
    return (out1, msgx)


# speedup vs baseline: 5.4858x; 1.6048x over previous
import functools

import jax
import jax.numpy as jnp
from jax import lax
from jax.experimental import pallas as pl
from jax.experimental.pallas import tpu as pltpu
from jax.experimental.pallas import tpu_sc as plsc

N = 10000
E = 320000
F = 128
K = 16

NC = 2
NS = 16
NW = NC * NS
EPW = E // NW
C = 40
NCHUNK = EPW // C
RPS = 640

_NBLK = 1000
_EBLK = 6400


def _dot_t(a, w):
    return lax.dot_general(a, w, (((1,), (1,)), ((), ())),
                           preferred_element_type=jnp.float32)


def _node_body(x_ref, we_ref, be_ref, wv_ref, bv_ref, he_ref, hv_ref):
    xa = jnp.maximum(x_ref[...], 0.0)
    he = _dot_t(xa, we_ref[...]) + be_ref[...]
    he_ref[...] = jnp.maximum(he, 0.0)
    hv = _dot_t(xa, wv_ref[...]) + bv_ref[...]
    hv_ref[...] = jnp.maximum(hv, 0.0)


def _node_call(x, We, be, Wv, bv):
    grid = (N // _NBLK,)
    return pl.pallas_call(
        _node_body,
        grid=grid,
        in_specs=[
            pl.BlockSpec((_NBLK, F), lambda i: (i, 0)),
            pl.BlockSpec((F, F), lambda i: (0, 0)),
            pl.BlockSpec((1, F), lambda i: (0, 0)),
            pl.BlockSpec((F, F), lambda i: (0, 0)),
            pl.BlockSpec((1, F), lambda i: (0, 0)),
        ],
        out_specs=[
            pl.BlockSpec((_NBLK, F), lambda i: (i, 0)),
            pl.BlockSpec((_NBLK, F), lambda i: (i, 0)),
        ],
        out_shape=[
            jax.ShapeDtypeStruct((N, F), jnp.float32),
            jax.ShapeDtypeStruct((N, F), jnp.float32),
        ],
    )(x, We, be.reshape(1, F), Wv, bv.reshape(1, F))


def _gate_body(eat_ref, wg_ref, gate_ref):
    gate_ref[...] = lax.dot_general(
        eat_ref[...], wg_ref[...], (((0,), (1,)), ((), ())),
        preferred_element_type=jnp.float32)


def _gate_call(edge_attr, WG):
    grid = (E // _EBLK,)
    return pl.pallas_call(
        _gate_body,
        grid=grid,
        in_specs=[
            pl.BlockSpec((K, _EBLK), lambda i: (0, i)),
            pl.BlockSpec((F, K), lambda i: (0, 0)),
        ],
        out_specs=pl.BlockSpec((_EBLK, F), lambda i: (i, 0)),
        out_shape=jax.ShapeDtypeStruct((E, F), jnp.float32),
    )(edge_attr.T, WG)


def _edge_sc_body(h_hbm, gate_hbm, src_hbm, dst_hbm, out_hbm,
                  src_all, dst_all, rows0, rows1, gate0, gate1,
                  acc_sh, sem_g0, sem_g1, sem_r0, sem_r1):
    core = lax.axis_index("core")
    sid = lax.axis_index("subcore")
    wid = sid * NC + core

    rows = (rows0, rows1)
    gate = (gate0, gate1)
    sem_g = (sem_g0, sem_g1)
    sem_r = (sem_r0, sem_r1)

    for buf in (*rows, *gate):
        @pl.loop(0, C)
        def _(r, buf=buf):
            for j in range(F // 16):
                buf[r, pl.ds(j * 16, 16)] = jnp.zeros((16,), jnp.float32)

    @pl.loop(0, RPS // (4 * C))
    def _(k):
        for j, buf in enumerate((*rows, *gate)):
            off = sid * RPS + (k * 4 + j) * C

            @pl.when(off + C <= N)
            def _(buf=buf, off=off):
                pltpu.sync_copy(buf, acc_sh.at[pl.ds(off, C)])

    pltpu.sync_copy(src_hbm.at[pl.ds(wid * EPW, EPW)], src_all)
    pltpu.sync_copy(dst_hbm.at[pl.ds(wid * EPW, EPW)], dst_all)

    plsc.subcore_barrier()

    def issue(i, b):
        base = wid * EPW + i * C
        pltpu.async_copy(gate_hbm.at[pl.ds(base, C)], gate[b], sem_g[b])
        pltpu.async_copy(h_hbm.at[src_all.at[pl.ds(i * C, C)]], rows[b],
                         sem_r[b])

    def consume(i, b):
        pltpu.make_async_copy(gate_hbm.at[pl.ds(0, C)], gate[b],
                              sem_g[b]).wait()
        pltpu.make_async_copy(h_hbm.at[src_all.at[pl.ds(0, C)]], rows[b],
                              sem_r[b]).wait()

        @pl.loop(0, C)
        def _(r):
            for j in range(F // 16):
                sl = pl.ds(j * 16, 16)
                rows[b][r, sl] = rows[b][r, sl] * gate[b][r, sl]

        pltpu.sync_copy(rows[b], acc_sh.at[dst_all.at[pl.ds(i * C, C)]],
                        add=True)

    issue(0, 0)
    issue(1, 1)

    @pl.loop(0, NCHUNK // 2 - 1)
    def _(t):
        i0 = 2 * t
        consume(i0, 0)
        issue(i0 + 2, 0)
        consume(i0 + 1, 1)
        issue(i0 + 3, 1)

    consume(NCHUNK - 2, 0)
    consume(NCHUNK - 1, 1)

    plsc.subcore_barrier()

    @pl.when(sid < NS - 1)
    def _():
        pltpu.sync_copy(acc_sh.at[pl.ds(sid * RPS, RPS)],
                        out_hbm.at[core, pl.ds(sid * RPS, RPS)])

    @pl.when(sid == NS - 1)
    def _():
        pltpu.sync_copy(acc_sh.at[pl.ds((NS - 1) * RPS, N - (NS - 1) * RPS)],
                        out_hbm.at[core, pl.ds((NS - 1) * RPS,
                                               N - (NS - 1) * RPS)])


def _edge_sc_call(h_e, gate, src, dst):
    mesh = plsc.VectorSubcoreMesh(core_axis_name="core",
                                  subcore_axis_name="subcore")
    k = pl.kernel(
        _edge_sc_body,
        out_type=jax.ShapeDtypeStruct((NC, N, F), jnp.float32),
        mesh=mesh,
        scratch_types=[
            pltpu.VMEM((EPW,), jnp.int32),
            pltpu.VMEM((EPW,), jnp.int32),
            pltpu.VMEM((C, F), jnp.float32),
            pltpu.VMEM((C, F), jnp.float32),
            pltpu.VMEM((C, F), jnp.float32),
            pltpu.VMEM((C, F), jnp.float32),
            pltpu.VMEM_SHARED((N, F), jnp.float32),
            pltpu.SemaphoreType.DMA,
            pltpu.SemaphoreType.DMA,
            pltpu.SemaphoreType.DMA,
            pltpu.SemaphoreType.DMA,
        ],
    )
    return k(h_e, gate, src, dst)


def _post_body(p_ref, hv_ref, x_ref, u_ref, wr1_ref, br1_ref, wr2_ref,
               br2_ref, wout_ref, bout_ref, out1_ref, out2_ref):
    aggr = p_ref[0] + p_ref[1]
    msgx = hv_ref[...] + aggr
    out2_ref[...] = msgx
    tmp = msgx
    for i in range(2):
        h = jnp.maximum(tmp, 0.0)
        h = jnp.maximum(_dot_t(h, wr1_ref[i]) + br1_ref[i], 0.0)
        h = _dot_t(h, wr2_ref[i]) + br2_ref[i]
        tmp = tmp + h
    v = _dot_t(tmp, wout_ref[...]) + bout_ref[...]
    out1_ref[...] = v + x_ref[...] * u_ref[...]


def _post_call(partials, h_v, x, u, Wr1, br1, Wr2, br2, Wout, bout):
    grid = (N // _NBLK,)
    return pl.pallas_call(
        _post_body,
        grid=grid,
        in_specs=[
            pl.BlockSpec((NC, _NBLK, F), lambda i: (0, i, 0)),
            pl.BlockSpec((_NBLK, F), lambda i: (i, 0)),
            pl.BlockSpec((_NBLK, F), lambda i: (i, 0)),
            pl.BlockSpec((1, F), lambda i: (0, 0)),
            pl.BlockSpec((2, F, F), lambda i: (0, 0, 0)),
            pl.BlockSpec((2, 1, F), lambda i: (0, 0, 0)),
            pl.BlockSpec((2, F, F), lambda i: (0, 0, 0)),
            pl.BlockSpec((2, 1, F), lambda i: (0, 0, 0)),
            pl.BlockSpec((F, F), lambda i: (0, 0)),
            pl.BlockSpec((1, F), lambda i: (0, 0)),
        ],
        out_specs=[
            pl.BlockSpec((_NBLK, F), lambda i: (i, 0)),
            pl.BlockSpec((_NBLK, F), lambda i: (i, 0)),
        ],
        out_shape=[
            jax.ShapeDtypeStruct((N, F), jnp.float32),
            jax.ShapeDtypeStruct((N, F), jnp.float32),
        ],
    )(partials, h_v, x, u, Wr1, br1.reshape(2, 1, F), Wr2,
      br2.reshape(2, 1, F), Wout, bout.reshape(1, F))


def kernel(x, edge_index, edge_attr, Wv, bv, We, be, WG, u, Wr1, br1, Wr2,
           br2, Wout, bout):
    src = edge_index[0]
    dst = edge_index[1]
    h_e, h_v = _node_call(x, We, be, Wv, bv)
    gate = _gate_call(edge_attr, WG)
    partials = _edge_sc_call(h_e, gate, src, dst)
    out1, msgx = _post_call(partials, h_v, x, u, Wr1, br1, Wr2, br2, Wout,
                            bout)
    return (out1, msgx)


# speedup vs baseline: 5.8527x; 1.0669x over previous
import functools

import jax
import jax.numpy as jnp
from jax import lax
from jax.experimental import pallas as pl
from jax.experimental.pallas import tpu as pltpu
from jax.experimental.pallas import tpu_sc as plsc

N = 10000
E = 320000
F = 128
K = 16

NC = 2
NS = 16
NW = NC * NS
EPW = E // NW
C = 40
NCHUNK = EPW // C
RPS = 640

_NBLK = 1000
_EBLK = 6400


def _dot_t(a, w):
    return lax.dot_general(a, w, (((1,), (1,)), ((), ())),
                           preferred_element_type=jnp.float32)


def _node_body(x_ref, we_ref, be_ref, wv_ref, bv_ref, he_ref, hv_ref):
    xa = jnp.maximum(x_ref[...], 0.0)
    he = _dot_t(xa, we_ref[...]) + be_ref[...]
    he_ref[...] = jnp.maximum(he, 0.0)
    hv = _dot_t(xa, wv_ref[...]) + bv_ref[...]
    hv_ref[...] = jnp.maximum(hv, 0.0)


def _node_call(x, We, be, Wv, bv):
    grid = (N // _NBLK,)
    return pl.pallas_call(
        _node_body,
        grid=grid,
        in_specs=[
            pl.BlockSpec((_NBLK, F), lambda i: (i, 0)),
            pl.BlockSpec((F, F), lambda i: (0, 0)),
            pl.BlockSpec((1, F), lambda i: (0, 0)),
            pl.BlockSpec((F, F), lambda i: (0, 0)),
            pl.BlockSpec((1, F), lambda i: (0, 0)),
        ],
        out_specs=[
            pl.BlockSpec((_NBLK, F), lambda i: (i, 0)),
            pl.BlockSpec((_NBLK, F), lambda i: (i, 0)),
        ],
        out_shape=[
            jax.ShapeDtypeStruct((N, F), jnp.float32),
            jax.ShapeDtypeStruct((N, F), jnp.float32),
        ],
    )(x, We, be.reshape(1, F), Wv, bv.reshape(1, F))


def _gate_body(eat_ref, wg_ref, gate_ref):
    gate_ref[...] = lax.dot_general(
        eat_ref[...], wg_ref[...], (((0,), (1,)), ((), ())),
        preferred_element_type=jnp.float32)


def _gate_call(edge_attr, WG):
    grid = (E // _EBLK,)
    return pl.pallas_call(
        _gate_body,
        grid=grid,
        in_specs=[
            pl.BlockSpec((K, _EBLK), lambda i: (0, i)),
            pl.BlockSpec((F, K), lambda i: (0, 0)),
        ],
        out_specs=pl.BlockSpec((_EBLK, F), lambda i: (i, 0)),
        out_shape=jax.ShapeDtypeStruct((E, F), jnp.float32),
    )(edge_attr.T, WG)


def _edge_sc_body(h_hbm, gate_hbm, src_hbm, dst_hbm, out_hbm,
                  src_all, dst_all, rows0, rows1, rows2, gate0, gate1, gate2,
                  acc_sh, sem_g0, sem_g1, sem_g2, sem_r0, sem_r1, sem_r2,
                  sem_s0, sem_s1, sem_s2):
    core = lax.axis_index("core")
    sid = lax.axis_index("subcore")
    wid = sid * NC + core

    rows = (rows0, rows1, rows2)
    gate = (gate0, gate1, gate2)
    sem_g = (sem_g0, sem_g1, sem_g2)
    sem_r = (sem_r0, sem_r1, sem_r2)
    sem_s = (sem_s0, sem_s1, sem_s2)

    zbufs = (rows[0], rows[1], gate[0], gate[1])
    for buf in zbufs:
        @pl.loop(0, C)
        def _(r, buf=buf):
            for j in range(F // 16):
                buf[r, pl.ds(j * 16, 16)] = jnp.zeros((16,), jnp.float32)

    @pl.loop(0, RPS // (4 * C))
    def _(k):
        for j, buf in enumerate(zbufs):
            off = sid * RPS + (k * 4 + j) * C

            @pl.when(off + C <= N)
            def _(buf=buf, off=off):
                pltpu.sync_copy(buf, acc_sh.at[pl.ds(off, C)])

    pltpu.sync_copy(src_hbm.at[pl.ds(wid * EPW, EPW)], src_all)
    pltpu.sync_copy(dst_hbm.at[pl.ds(wid * EPW, EPW)], dst_all)

    plsc.subcore_barrier()

    def issue(i, b, drain):
        if drain:
            pltpu.make_async_copy(rows[b],
                                  acc_sh.at[dst_all.at[pl.ds(0, C)]],
                                  sem_s[b]).wait()
        base = wid * EPW + i * C
        pltpu.async_copy(gate_hbm.at[pl.ds(base, C)], gate[b], sem_g[b])
        pltpu.async_copy(h_hbm.at[src_all.at[pl.ds(i * C, C)]], rows[b],
                         sem_r[b])

    def consume(i, b):
        pltpu.make_async_copy(gate_hbm.at[pl.ds(0, C)], gate[b],
                              sem_g[b]).wait()
        pltpu.make_async_copy(h_hbm.at[src_all.at[pl.ds(0, C)]], rows[b],
                              sem_r[b]).wait()

        @pl.loop(0, C, step=4)
        def _(r):
            for rr in range(4):
                for j in range(F // 16):
                    sl = pl.ds(j * 16, 16)
                    rows[b][r + rr, sl] = (rows[b][r + rr, sl]
                                           * gate[b][r + rr, sl])

        pltpu.async_copy(rows[b], acc_sh.at[dst_all.at[pl.ds(i * C, C)]],
                         sem_s[b], add=True)

    def step(j, with_issue):
        consume(j, j % 3)
        if with_issue:
            issue(j + 2, (j + 2) % 3, drain=j >= 1)

    issue(0, 0, False)
    issue(1, 1, False)
    step(0, True)
    step(1, True)
    step(2, True)

    @pl.loop(1, (NCHUNK - 4) // 3)
    def _(t):
        for k in range(3):
            j = 3 * t + k
            consume(j, k)
            issue(j + 2, (k + 2) % 3, drain=True)

    step(NCHUNK - 4, True)
    step(NCHUNK - 3, True)
    step(NCHUNK - 2, False)
    step(NCHUNK - 1, False)

    for b in range(3):
        pltpu.make_async_copy(rows[b], acc_sh.at[dst_all.at[pl.ds(0, C)]],
                              sem_s[b]).wait()

    plsc.subcore_barrier()

    @pl.when(sid < NS - 1)
    def _():
        pltpu.sync_copy(acc_sh.at[pl.ds(sid * RPS, RPS)],
                        out_hbm.at[core, pl.ds(sid * RPS, RPS)])

    @pl.when(sid == NS - 1)
    def _():
        pltpu.sync_copy(acc_sh.at[pl.ds((NS - 1) * RPS, N - (NS - 1) * RPS)],
                        out_hbm.at[core, pl.ds((NS - 1) * RPS,
                                               N - (NS - 1) * RPS)])


def _edge_sc_call(h_e, gate, src, dst):
    mesh = plsc.VectorSubcoreMesh(core_axis_name="core",
                                  subcore_axis_name="subcore")
    k = pl.kernel(
        _edge_sc_body,
        out_type=jax.ShapeDtypeStruct((NC, N, F), jnp.float32),
        mesh=mesh,
        scratch_types=[
            pltpu.VMEM((EPW,), jnp.int32),
            pltpu.VMEM((EPW,), jnp.int32),
            pltpu.VMEM((C, F), jnp.float32),
            pltpu.VMEM((C, F), jnp.float32),
            pltpu.VMEM((C, F), jnp.float32),
            pltpu.VMEM((C, F), jnp.float32),
            pltpu.VMEM((C, F), jnp.float32),
            pltpu.VMEM((C, F), jnp.float32),
            pltpu.VMEM_SHARED((N, F), jnp.float32),
            pltpu.SemaphoreType.DMA,
            pltpu.SemaphoreType.DMA,
            pltpu.SemaphoreType.DMA,
            pltpu.SemaphoreType.DMA,
            pltpu.SemaphoreType.DMA,
            pltpu.SemaphoreType.DMA,
            pltpu.SemaphoreType.DMA,
            pltpu.SemaphoreType.DMA,
            pltpu.SemaphoreType.DMA,
        ],
    )
    return k(h_e, gate, src, dst)


def _post_body(p_ref, hv_ref, x_ref, u_ref, wr1_ref, br1_ref, wr2_ref,
               br2_ref, wout_ref, bout_ref, out1_ref, out2_ref):
    aggr = p_ref[0] + p_ref[1]
    msgx = hv_ref[...] + aggr
    out2_ref[...] = msgx
    tmp = msgx
    for i in range(2):
        h = jnp.maximum(tmp, 0.0)
        h = jnp.maximum(_dot_t(h, wr1_ref[i]) + br1_ref[i], 0.0)
        h = _dot_t(h, wr2_ref[i]) + br2_ref[i]
        tmp = tmp + h
    v = _dot_t(tmp, wout_ref[...]) + bout_ref[...]
    out1_ref[...] = v + x_ref[...] * u_ref[...]


def _post_call(partials, h_v, x, u, Wr1, br1, Wr2, br2, Wout, bout):
    grid = (N // _NBLK,)
    return pl.pallas_call(
        _post_body,
        grid=grid,
        in_specs=[
            pl.BlockSpec((NC, _NBLK, F), lambda i: (0, i, 0)),
            pl.BlockSpec((_NBLK, F), lambda i: (i, 0)),
            pl.BlockSpec((_NBLK, F), lambda i: (i, 0)),
            pl.BlockSpec((1, F), lambda i: (0, 0)),
            pl.BlockSpec((2, F, F), lambda i: (0, 0, 0)),
            pl.BlockSpec((2, 1, F), lambda i: (0, 0, 0)),
            pl.BlockSpec((2, F, F), lambda i: (0, 0, 0)),
            pl.BlockSpec((2, 1, F), lambda i: (0, 0, 0)),
            pl.BlockSpec((F, F), lambda i: (0, 0)),
            pl.BlockSpec((1, F), lambda i: (0, 0)),
        ],
        out_specs=[
            pl.BlockSpec((_NBLK, F), lambda i: (i, 0)),
            pl.BlockSpec((_NBLK, F), lambda i: (i, 0)),
        ],
        out_shape=[
            jax.ShapeDtypeStruct((N, F), jnp.float32),
            jax.ShapeDtypeStruct((N, F), jnp.float32),
        ],
    )(partials, h_v, x, u, Wr1, br1.reshape(2, 1, F), Wr2,
      br2.reshape(2, 1, F), Wout, bout.reshape(1, F))


def kernel(x, edge_index, edge_attr, Wv, bv, We, be, WG, u, Wr1, br1, Wr2,
           br2, Wout, bout):
    src = edge_index[0]
    dst = edge_index[1]
    h_e, h_v = _node_call(x, We, be, Wv, bv)
    gate = _gate_call(edge_attr, WG)
    partials = _edge_sc_call(h_e, gate, src, dst)
    out1, msgx = _post_call(partials, h_v, x, u, Wr1, br1, Wr2, br2, Wout,
                            bout)
    return (out1, msgx)
